# baseline (speedup 1.0000x reference)
import functools

import jax
import jax.numpy as jnp
from jax import lax
from jax.experimental import pallas as pl
from jax.experimental.pallas import tpu as pltpu
from jax.experimental.pallas import tpu_sc as plsc

N = 10000
NPAD = 10240
D = 128
OUT = 64
NGRAPH = 64
K = 30
E = 320000
NC, NS = 2, 16
NW = NC * NS
CHUNK = 100
NCHUNK = 100
RPT = NPAD // NS
NEG = -3.4e38
BIGI = 2 ** 30

_SC_MESH = plsc.VectorSubcoreMesh(
    core_axis_name="c", subcore_axis_name="s", num_cores=NC, num_subcores=NS)
_SC_PARAMS = pltpu.CompilerParams(use_tc_tiling_on_sc=False)


def _make_hist():
    @functools.partial(
        pl.kernel,
        out_type=[jax.ShapeDtypeStruct((NC, NPAD, 16), jnp.float32),
                  jax.ShapeDtypeStruct((NC, 128, 16), jnp.float32)],
        mesh=_SC_MESH,
        compiler_params=_SC_PARAMS,
        scratch_types=[
            pltpu.VMEM((NCHUNK, CHUNK), jnp.int32),
            pltpu.VMEM((4, 80), jnp.int32),
            pltpu.VMEM((CHUNK, 16), jnp.float32),
            pltpu.VMEM_SHARED((NPAD, 16), jnp.float32),
            pltpu.VMEM_SHARED((128, 16), jnp.float32),
        ],
    )
    def hist(dsts_hbm, batch_hbm, zeros_hbm, ones_hbm, out_hbm, gout_hbm,
             dst_v, bat_v, ones_v, cnt, gcnt):
        cid = lax.axis_index("c")
        sid = lax.axis_index("s")
        wid = cid * NS + sid
        pltpu.sync_copy(zeros_hbm, cnt.at[pl.ds(sid * RPT, RPT)])

        @pl.when(sid == 0)
        def _():
            pltpu.sync_copy(zeros_hbm.at[pl.ds(0, 128)], gcnt)

        pltpu.sync_copy(ones_hbm, ones_v)
        pltpu.sync_copy(dsts_hbm.at[wid], dst_v)
        pltpu.sync_copy(batch_hbm.at[wid], bat_v)
        plsc.subcore_barrier()

        def body(c, carry):
            pltpu.sync_copy(ones_v, cnt.at[dst_v.at[c]], add=True)
            return carry

        lax.fori_loop(0, NCHUNK, body, 0)
        for j in range(4):
            pltpu.sync_copy(ones_v.at[pl.ds(0, 80)], gcnt.at[bat_v.at[j]],
                            add=True)
        plsc.subcore_barrier()
        pltpu.sync_copy(cnt.at[pl.ds(sid * RPT, RPT)],
                        out_hbm.at[cid, pl.ds(sid * RPT, RPT)])

        @pl.when(sid == 0)
        def _():
            pltpu.sync_copy(gcnt, gout_hbm.at[cid])

    return hist


def _make_edge_scatter(width):
    @functools.partial(
        pl.kernel,
        out_type=jax.ShapeDtypeStruct((NC, NPAD, width), jnp.float32),
        mesh=_SC_MESH,
        compiler_params=_SC_PARAMS,
        scratch_types=[
            pltpu.VMEM((NCHUNK, CHUNK), jnp.int32),
            pltpu.VMEM((NCHUNK, CHUNK), jnp.int32),
            pltpu.VMEM((CHUNK, width), jnp.float32),
            pltpu.VMEM((CHUNK, width), jnp.float32),
            pltpu.VMEM_SHARED((NPAD, width), jnp.float32),
            pltpu.SemaphoreType.DMA,
            pltpu.SemaphoreType.DMA,
        ],
    )
    def scat(g_hbm, srcs_hbm, dsts_hbm, zeros_hbm, out_hbm,
             src_v, dst_v, buf0, buf1, acc, sem0, sem1):
        cid = lax.axis_index("c")
        sid = lax.axis_index("s")
        wid = cid * NS + sid
        pltpu.sync_copy(zeros_hbm, acc.at[pl.ds(sid * RPT, RPT)])
        pltpu.sync_copy(srcs_hbm.at[wid], src_v)
        pltpu.sync_copy(dsts_hbm.at[wid], dst_v)
        plsc.subcore_barrier()

        pltpu.async_copy(g_hbm.at[src_v.at[0]], buf0, sem0)

        def body(it, carry):
            c = 2 * it
            pltpu.async_copy(g_hbm.at[src_v.at[c + 1]], buf1, sem1)
            pltpu.make_async_copy(g_hbm.at[src_v.at[c]], buf0, sem0).wait()
            pltpu.sync_copy(buf0, acc.at[dst_v.at[c]], add=True)

            @pl.when(c + 2 < NCHUNK)
            def _():
                pltpu.async_copy(g_hbm.at[src_v.at[c + 2]], buf0, sem0)

            pltpu.make_async_copy(g_hbm.at[src_v.at[c + 1]], buf1, sem1).wait()
            pltpu.sync_copy(buf1, acc.at[dst_v.at[c + 1]], add=True)
            return carry

        lax.fori_loop(0, NCHUNK // 2, body, 0)
        plsc.subcore_barrier()
        pltpu.sync_copy(acc.at[pl.ds(sid * RPT, RPT)],
                        out_hbm.at[cid, pl.ds(sid * RPT, RPT)])

    return scat


WC = 5
WIN = WC * 128
WMAX = WIN - 8


def _make_win_gather():
    @functools.partial(
        pl.kernel,
        out_type=jax.ShapeDtypeStruct((NGRAPH, WIN), jnp.float32),
        mesh=_SC_MESH,
        compiler_params=_SC_PARAMS,
        scratch_types=[
            pltpu.VMEM((128,), jnp.int32),
            pltpu.VMEM((128,), jnp.float32),
            pltpu.SemaphoreType.DMA,
        ],
    )
    def wgat(last_hbm, widx_hbm, out_hbm, idx_v, buf, sem):
        cid = lax.axis_index("c")
        sid = lax.axis_index("s")
        wid = cid * NS + sid
        for j in range(2):
            g = 2 * wid + j
            for c in range(WC):
                pltpu.sync_copy(widx_hbm.at[c, g], idx_v)
                pltpu.async_copy(last_hbm.at[idx_v], buf, sem).wait()
                pltpu.sync_copy(buf, out_hbm.at[g, pl.ds(c * 128, 128)])

    return wgat


def _make_row_gather(n_per_w):
    @functools.partial(
        pl.kernel,
        out_type=jax.ShapeDtypeStruct((NW * n_per_w, OUT), jnp.float32),
        mesh=_SC_MESH,
        compiler_params=_SC_PARAMS,
        scratch_types=[
            pltpu.VMEM((n_per_w,), jnp.int32),
            pltpu.VMEM((n_per_w, OUT), jnp.float32),
            pltpu.SemaphoreType.DMA,
        ],
    )
    def gat(h3_hbm, idx_hbm, out_hbm, idx_v, rows_v, sem):
        cid = lax.axis_index("c")
        sid = lax.axis_index("s")
        wid = cid * NS + sid
        pltpu.sync_copy(idx_hbm.at[wid], idx_v)
        pltpu.async_copy(h3_hbm.at[idx_v], rows_v, sem).wait()
        pltpu.sync_copy(rows_v, out_hbm.at[pl.ds(wid * n_per_w, n_per_w)])

    return gat


_BLK = 1024
_GRID = NPAD // _BLK


def _a0_kernel(x_ref, w_ref, p_ref, g_ref, dv_ref):
    deg = 1.0 + p_ref[0, :, 0:1] + p_ref[1, :, 0:1]
    dv = jnp.broadcast_to(1.0 / jnp.sqrt(deg), (_BLK, D))
    dv_ref[...] = dv
    xw = jnp.dot(x_ref[...], w_ref[...], preferred_element_type=jnp.float32)
    g_ref[...] = dv * xw


def _a0(x, w, parts):
    return pl.pallas_call(
        _a0_kernel,
        grid=(_GRID,),
        in_specs=[
            pl.BlockSpec((_BLK, D), lambda i: (i, 0)),
            pl.BlockSpec((D, D), lambda i: (0, 0)),
            pl.BlockSpec((NC, _BLK, 16), lambda i: (0, i, 0)),
        ],
        out_specs=[pl.BlockSpec((_BLK, D), lambda i: (i, 0)),
                   pl.BlockSpec((_BLK, D), lambda i: (i, 0))],
        out_shape=[jax.ShapeDtypeStruct((NPAD, D), jnp.float32),
                   jax.ShapeDtypeStruct((NPAD, D), jnp.float32)],
    )(x, w, parts)


def _make_a_mid(wout):
    def a_kernel(acc_ref, g_ref, dv_ref, b_ref, w_ref, o_ref):
        i = pl.program_id(0)
        rows = i * _BLK + lax.broadcasted_iota(jnp.int32, (_BLK, 1), 0)
        t = dv_ref[...] * (acc_ref[0] + acc_ref[1] + g_ref[...]) + b_ref[...]
        t = jnp.where(rows < N, jax.nn.relu(t), 0.0)
        tw = jnp.dot(t, w_ref[...], preferred_element_type=jnp.float32)
        o_ref[...] = dv_ref[:, :wout] * tw

    def run(acc, g, dinvb, b, w):
        return pl.pallas_call(
            a_kernel,
            grid=(_GRID,),
            in_specs=[
                pl.BlockSpec((NC, _BLK, D), lambda i: (0, i, 0)),
                pl.BlockSpec((_BLK, D), lambda i: (i, 0)),
                pl.BlockSpec((_BLK, D), lambda i: (i, 0)),
                pl.BlockSpec((1, D), lambda i: (0, 0)),
                pl.BlockSpec((D, wout), lambda i: (0, 0)),
            ],
            out_specs=pl.BlockSpec((_BLK, wout), lambda i: (i, 0)),
            out_shape=jax.ShapeDtypeStruct((NPAD, wout), jnp.float32),
        )(acc, g, dinvb, b, w)

    return run


def _prep_kernel(acc_ref, g_ref, dv_ref, b_ref, gp_ref,
                 h3_ref, last_ref, widx_ref, lo_ref, hi_ref, ws_ref, ovf_ref):
    rows = lax.broadcasted_iota(jnp.int32, (NPAD, 1), 0)
    t = dv_ref[:, :OUT] * (acc_ref[0] + acc_ref[1] + g_ref[...]) + b_ref[...]
    h3 = jnp.where(rows < N, t, 0.0)
    h3_ref[...] = h3
    last_ref[...] = h3[:, OUT - 1:OUT]

    cnt = gp_ref[0, :NGRAPH, 0:1] + gp_ref[1, :NGRAPH, 0:1]
    gi = lax.broadcasted_iota(jnp.int32, (NGRAPH, NGRAPH), 0)
    gj = lax.broadcasted_iota(jnp.int32, (NGRAPH, NGRAPH), 1)
    lower = jnp.where(gi > gj, 1.0, 0.0)
    starts = jnp.dot(lower, cnt, preferred_element_type=jnp.float32)
    starts_i = starts.astype(jnp.int32)
    cnt_i = cnt.astype(jnp.int32)
    ws = jnp.bitwise_and(starts_i, -8)
    lo_ref[...] = starts_i - ws
    hi_ref[...] = starts_i - ws + cnt_i
    ws_ref[...] = ws
    ovf_ref[...] = jnp.max(cnt_i, keepdims=True).reshape(1, 1) > WMAX
    lane = lax.broadcasted_iota(jnp.int32, (NGRAPH, 128), 1)
    for c in range(WC):
        widx_ref[c] = jnp.minimum(ws + c * 128 + lane, NPAD - 1)


def _prep(acc, g, dinvb, b, gparts):
    return pl.pallas_call(
        _prep_kernel,
        in_specs=[
            pl.BlockSpec((NC, NPAD, OUT), lambda: (0, 0, 0)),
            pl.BlockSpec((NPAD, OUT), lambda: (0, 0)),
            pl.BlockSpec((NPAD, D), lambda: (0, 0)),
            pl.BlockSpec((1, OUT), lambda: (0, 0)),
            pl.BlockSpec((NC, 128, 16), lambda: (0, 0, 0)),
        ],
        out_specs=[pl.BlockSpec((NPAD, OUT), lambda: (0, 0)),
                   pl.BlockSpec((NPAD, 1), lambda: (0, 0)),
                   pl.BlockSpec((WC, NGRAPH, 128), lambda: (0, 0, 0)),
                   pl.BlockSpec((NGRAPH, 1), lambda: (0, 0)),
                   pl.BlockSpec((NGRAPH, 1), lambda: (0, 0)),
                   pl.BlockSpec((NGRAPH, 1), lambda: (0, 0)),
                   pl.BlockSpec((1, 1), lambda: (0, 0))],
        out_shape=[jax.ShapeDtypeStruct((NPAD, OUT), jnp.float32),
                   jax.ShapeDtypeStruct((NPAD, 1), jnp.float32),
                   jax.ShapeDtypeStruct((WC, NGRAPH, 128), jnp.int32),
                   jax.ShapeDtypeStruct((NGRAPH, 1), jnp.int32),
                   jax.ShapeDtypeStruct((NGRAPH, 1), jnp.int32),
                   jax.ShapeDtypeStruct((NGRAPH, 1), jnp.int32),
                   jax.ShapeDtypeStruct((1, 1), jnp.bool_)],
    )(acc, g, dinvb, b, gparts)


def _select_kernel(win_ref, lo_ref, hi_ref, ws_ref, idx_ref, scores_ref):
    lane = lax.broadcasted_iota(jnp.int32, (NGRAPH, WIN), 1)
    valid = (lane >= lo_ref[...]) & (lane < hi_ref[...])
    scores_ref[...] = jnp.where(valid, win_ref[...], NEG)
    tcol = lax.broadcasted_iota(jnp.int32, (NGRAPH, 32), 1)

    def body(t, carry):
        prev_ix, acc_idx = carry
        s = jnp.where(lane == prev_ix, NEG, scores_ref[...])
        scores_ref[...] = s
        m = jnp.max(s, axis=1, keepdims=True)
        cand = jnp.where(s >= m, lane, BIGI)
        ix = jnp.min(cand, axis=1, keepdims=True)
        node = jnp.where(m > -1e38, ws_ref[...] + ix, N)
        acc_idx = jnp.where(tcol == t, node, acc_idx)
        return ix, acc_idx

    _, acc_idx = lax.fori_loop(
        0, 32, body,
        (jnp.full((NGRAPH, 1), -1, jnp.int32),
         jnp.zeros((NGRAPH, 32), jnp.int32)))
    idx_ref[...] = acc_idx


def _select(win, lo, hi, ws):
    return pl.pallas_call(
        _select_kernel,
        in_specs=[
            pl.BlockSpec((NGRAPH, WIN), lambda: (0, 0)),
            pl.BlockSpec((NGRAPH, 1), lambda: (0, 0)),
            pl.BlockSpec((NGRAPH, 1), lambda: (0, 0)),
            pl.BlockSpec((NGRAPH, 1), lambda: (0, 0)),
        ],
        out_specs=pl.BlockSpec((NGRAPH, 32), lambda: (0, 0)),
        out_shape=jax.ShapeDtypeStruct((NGRAPH, 32), jnp.int32),
        scratch_shapes=[pltpu.VMEM((NGRAPH, WIN), jnp.float32)],
    )(win, lo, hi, ws)


def _topk_full_kernel(h3_ref, batch_ref, idx_ref, scores_ref):
    iota_g = lax.broadcasted_iota(jnp.int32, (NPAD, NGRAPH), 1)
    iota_n = lax.broadcasted_iota(jnp.int32, (NPAD, NGRAPH), 0)
    last = h3_ref[:, OUT - 1:OUT]
    eq = batch_ref[...] == iota_g
    scores_ref[...] = jnp.where(eq, jnp.broadcast_to(last, (NPAD, NGRAPH)), NEG)

    def body(t, prev_ix):
        s = jnp.where(iota_n == prev_ix, NEG, scores_ref[...])
        scores_ref[...] = s
        m = jnp.max(s, axis=0, keepdims=True)
        cand = jnp.where(s >= m, iota_n, BIGI)
        ix = jnp.min(cand, axis=0, keepdims=True)
        idx_ref[pl.ds(t, 1), :] = jnp.where(m > -1e38, ix, N)
        return ix

    lax.fori_loop(0, 32, body, jnp.full((1, NGRAPH), -1, jnp.int32))


def _topk_full(h3, batch2d):
    return pl.pallas_call(
        _topk_full_kernel,
        in_specs=[
            pl.BlockSpec((NPAD, OUT), lambda: (0, 0)),
            pl.BlockSpec((NPAD, 1), lambda: (0, 0)),
        ],
        out_specs=pl.BlockSpec((32, NGRAPH), lambda: (0, 0)),
        out_shape=jax.ShapeDtypeStruct((32, NGRAPH), jnp.int32),
        scratch_shapes=[pltpu.VMEM((NPAD, NGRAPH), jnp.float32)],
    )(h3, batch2d)


def kernel(x, edge_index, batch, W0, b0, W1, b1, W2, b2):
    xp = jnp.pad(x, ((0, NPAD - N), (0, 0)))
    npad_tail = NPAD - N
    srcs = edge_index[0].reshape(NW, NCHUNK, CHUNK)
    dsts = edge_index[1].reshape(NW, NCHUNK, CHUNK)
    batch2d = jnp.pad(batch, (0, NPAD - N), constant_values=NGRAPH)[:, None]
    z16 = jnp.zeros((RPT, 16), jnp.float32)
    z128 = jnp.zeros((RPT, D), jnp.float32)
    z64 = jnp.zeros((RPT, OUT), jnp.float32)
    ones16 = jnp.ones((CHUNK, 16), jnp.float32)

    batch_pad = jnp.concatenate(
        [batch, NGRAPH + (jnp.arange(NPAD - N, dtype=jnp.int32) % NGRAPH)])
    batch_chunks = batch_pad.reshape(NW, 4, 80)

    deg_parts, g_parts = _make_hist()(dsts, batch_chunks, z16, ones16)

    scat128 = _make_edge_scatter(D)
    g0, dinvb = _a0(xp, W0, deg_parts)
    acc0 = scat128(g0, srcs, dsts, z128)
    g1 = _make_a_mid(D)(acc0, g0, dinvb, b0.reshape(1, D), W1)
    acc1 = scat128(g1, srcs, dsts, z128)
    g2 = _make_a_mid(OUT)(acc1, g1, dinvb, b1.reshape(1, D), W2)
    acc2 = _make_edge_scatter(OUT)(g2, srcs, dsts, z64)

    h3, lastc, widx, lo, hi, ws, ovf = _prep(
        acc2, g2, dinvb, b2.reshape(1, OUT), g_parts)
    win = _make_win_gather()(lastc.reshape(NPAD), widx)

    def fast(_):
        return _select(win, lo, hi, ws)

    def slow(_):
        return _topk_full(h3, batch2d).T

    idx_gk = lax.cond(ovf[0, 0], slow, fast, 0)
    idx_flat = idx_gk[:, :K].reshape(-1)
    n_per_w = 64
    dummy = N + (jnp.arange(NW * n_per_w - NGRAPH * K, dtype=jnp.int32) % npad_tail)
    idx_pad = jnp.concatenate([idx_flat, dummy]).reshape(NW, n_per_w)
    rows = _make_row_gather(n_per_w)(h3, idx_pad)
    return rows[:NGRAPH * K].reshape(NGRAPH, K * OUT)


# speedup vs baseline: 1.0018x; 1.0018x over previous
import functools

import jax
import jax.numpy as jnp
from jax import lax
from jax.experimental import pallas as pl
from jax.experimental.pallas import tpu as pltpu
from jax.experimental.pallas import tpu_sc as plsc

N = 10000
NPAD = 10240
D = 128
OUT = 64
NGRAPH = 64
K = 30
E = 320000
NC, NS = 2, 16
NW = NC * NS
CHUNK = 100
NCHUNK = 100
RPT = NPAD // NS
NEG = -3.4e38
BIGI = 2 ** 30

_SC_MESH = plsc.VectorSubcoreMesh(
    core_axis_name="c", subcore_axis_name="s", num_cores=NC, num_subcores=NS)
_SC_PARAMS = pltpu.CompilerParams(use_tc_tiling_on_sc=False)


def _make_hist():
    @functools.partial(
        pl.kernel,
        out_type=[jax.ShapeDtypeStruct((NC, NPAD, 16), jnp.float32),
                  jax.ShapeDtypeStruct((NC, 128, 16), jnp.float32)],
        mesh=_SC_MESH,
        compiler_params=_SC_PARAMS,
        scratch_types=[
            pltpu.VMEM((NCHUNK, CHUNK), jnp.int32),
            pltpu.VMEM((4, 80), jnp.int32),
            pltpu.VMEM((CHUNK, 16), jnp.float32),
            pltpu.VMEM_SHARED((NPAD, 16), jnp.float32),
            pltpu.VMEM_SHARED((128, 16), jnp.float32),
        ],
    )
    def hist(dsts_hbm, batch_hbm, zeros_hbm, ones_hbm, out_hbm, gout_hbm,
             dst_v, bat_v, ones_v, cnt, gcnt):
        cid = lax.axis_index("c")
        sid = lax.axis_index("s")
        wid = cid * NS + sid
        pltpu.sync_copy(zeros_hbm, cnt.at[pl.ds(sid * RPT, RPT)])

        @pl.when(sid == 0)
        def _():
            pltpu.sync_copy(zeros_hbm.at[pl.ds(0, 128)], gcnt)

        pltpu.sync_copy(ones_hbm, ones_v)
        pltpu.sync_copy(dsts_hbm.at[wid], dst_v)
        pltpu.sync_copy(batch_hbm.at[wid], bat_v)
        plsc.subcore_barrier()

        def body(c, carry):
            pltpu.sync_copy(ones_v, cnt.at[dst_v.at[c]], add=True)
            return carry

        lax.fori_loop(0, NCHUNK, body, 0)
        for j in range(4):
            pltpu.sync_copy(ones_v.at[pl.ds(0, 80)], gcnt.at[bat_v.at[j]],
                            add=True)
        plsc.subcore_barrier()
        pltpu.sync_copy(cnt.at[pl.ds(sid * RPT, RPT)],
                        out_hbm.at[cid, pl.ds(sid * RPT, RPT)])

        @pl.when(sid == 0)
        def _():
            pltpu.sync_copy(gcnt, gout_hbm.at[cid])

    return hist


def _make_edge_scatter(width):
    @functools.partial(
        pl.kernel,
        out_type=jax.ShapeDtypeStruct((NC, NPAD, width), jnp.float32),
        mesh=_SC_MESH,
        compiler_params=_SC_PARAMS,
        scratch_types=[
            pltpu.VMEM((NCHUNK, CHUNK), jnp.int32),
            pltpu.VMEM((NCHUNK, CHUNK), jnp.int32),
            pltpu.VMEM((CHUNK, width), jnp.float32),
            pltpu.VMEM((CHUNK, width), jnp.float32),
            pltpu.VMEM_SHARED((NPAD, width), jnp.float32),
            pltpu.SemaphoreType.DMA,
            pltpu.SemaphoreType.DMA,
        ],
    )
    def scat(g_hbm, srcs_hbm, dsts_hbm, zeros_hbm, out_hbm,
             src_v, dst_v, buf0, buf1, acc, sem0, sem1):
        cid = lax.axis_index("c")
        sid = lax.axis_index("s")
        wid = cid * NS + sid
        pltpu.sync_copy(zeros_hbm, acc.at[pl.ds(sid * RPT, RPT)])
        pltpu.sync_copy(srcs_hbm.at[wid], src_v)
        pltpu.sync_copy(dsts_hbm.at[wid], dst_v)
        plsc.subcore_barrier()

        pltpu.async_copy(g_hbm.at[src_v.at[0]], buf0, sem0)

        def body(it, carry):
            c = 2 * it
            pltpu.async_copy(g_hbm.at[src_v.at[c + 1]], buf1, sem1)
            pltpu.make_async_copy(g_hbm.at[src_v.at[c]], buf0, sem0).wait()
            pltpu.sync_copy(buf0, acc.at[dst_v.at[c]], add=True)

            @pl.when(c + 2 < NCHUNK)
            def _():
                pltpu.async_copy(g_hbm.at[src_v.at[c + 2]], buf0, sem0)

            pltpu.make_async_copy(g_hbm.at[src_v.at[c + 1]], buf1, sem1).wait()
            pltpu.sync_copy(buf1, acc.at[dst_v.at[c + 1]], add=True)
            return carry

        lax.fori_loop(0, NCHUNK // 2, body, 0)
        plsc.subcore_barrier()
        pltpu.sync_copy(acc.at[pl.ds(sid * RPT, RPT)],
                        out_hbm.at[cid, pl.ds(sid * RPT, RPT)])

    return scat


WC = 5
WIN = WC * 128
WMAX = WIN - 8


def _make_win_gather():
    @functools.partial(
        pl.kernel,
        out_type=jax.ShapeDtypeStruct((NGRAPH, WIN), jnp.float32),
        mesh=_SC_MESH,
        compiler_params=_SC_PARAMS,
        scratch_types=[
            pltpu.VMEM((128,), jnp.int32),
            pltpu.VMEM((128,), jnp.float32),
            pltpu.SemaphoreType.DMA,
        ],
    )
    def wgat(last_hbm, widx_hbm, out_hbm, idx_v, buf, sem):
        cid = lax.axis_index("c")
        sid = lax.axis_index("s")
        wid = cid * NS + sid
        for j in range(2):
            g = 2 * wid + j
            for c in range(WC):
                pltpu.sync_copy(widx_hbm.at[c, g], idx_v)
                pltpu.async_copy(last_hbm.at[idx_v], buf, sem).wait()
                pltpu.sync_copy(buf, out_hbm.at[g, pl.ds(c * 128, 128)])

    return wgat


def _make_row_gather(n_per_w):
    @functools.partial(
        pl.kernel,
        out_type=jax.ShapeDtypeStruct((NW * n_per_w, OUT), jnp.float32),
        mesh=_SC_MESH,
        compiler_params=_SC_PARAMS,
        scratch_types=[
            pltpu.VMEM((n_per_w,), jnp.int32),
            pltpu.VMEM((n_per_w, OUT), jnp.float32),
            pltpu.SemaphoreType.DMA,
        ],
    )
    def gat(h3_hbm, idx_hbm, out_hbm, idx_v, rows_v, sem):
        cid = lax.axis_index("c")
        sid = lax.axis_index("s")
        wid = cid * NS + sid
        pltpu.sync_copy(idx_hbm.at[wid], idx_v)
        pltpu.async_copy(h3_hbm.at[idx_v], rows_v, sem).wait()
        pltpu.sync_copy(rows_v, out_hbm.at[pl.ds(wid * n_per_w, n_per_w)])

    return gat


_BLK = 1024
_GRID = NPAD // _BLK


def _a0_kernel(x_ref, w_ref, p_ref, g_ref, dv_ref):
    deg = 1.0 + p_ref[0, :, 0:1] + p_ref[1, :, 0:1]
    dv = jnp.broadcast_to(lax.rsqrt(deg), (_BLK, D))
    dv_ref[...] = dv
    xw = jnp.dot(x_ref[...], w_ref[...], preferred_element_type=jnp.float32)
    g_ref[...] = dv * xw


def _a0(x, w, parts):
    return pl.pallas_call(
        _a0_kernel,
        grid=(_GRID,),
        in_specs=[
            pl.BlockSpec((_BLK, D), lambda i: (i, 0)),
            pl.BlockSpec((D, D), lambda i: (0, 0)),
            pl.BlockSpec((NC, _BLK, 16), lambda i: (0, i, 0)),
        ],
        out_specs=[pl.BlockSpec((_BLK, D), lambda i: (i, 0)),
                   pl.BlockSpec((_BLK, D), lambda i: (i, 0))],
        out_shape=[jax.ShapeDtypeStruct((NPAD, D), jnp.float32),
                   jax.ShapeDtypeStruct((NPAD, D), jnp.float32)],
    )(x, w, parts)


def _make_a_mid(wout):
    def a_kernel(acc_ref, g_ref, dv_ref, b_ref, w_ref, o_ref):
        i = pl.program_id(0)
        rows = i * _BLK + lax.broadcasted_iota(jnp.int32, (_BLK, 1), 0)
        t = dv_ref[...] * (acc_ref[0] + acc_ref[1] + g_ref[...]) + b_ref[...]
        t = jnp.where(rows < N, jax.nn.relu(t), 0.0)
        tw = jnp.dot(t, w_ref[...], preferred_element_type=jnp.float32)
        o_ref[...] = dv_ref[:, :wout] * tw

    def run(acc, g, dinvb, b, w):
        return pl.pallas_call(
            a_kernel,
            grid=(_GRID,),
            in_specs=[
                pl.BlockSpec((NC, _BLK, D), lambda i: (0, i, 0)),
                pl.BlockSpec((_BLK, D), lambda i: (i, 0)),
                pl.BlockSpec((_BLK, D), lambda i: (i, 0)),
                pl.BlockSpec((1, D), lambda i: (0, 0)),
                pl.BlockSpec((D, wout), lambda i: (0, 0)),
            ],
            out_specs=pl.BlockSpec((_BLK, wout), lambda i: (i, 0)),
            out_shape=jax.ShapeDtypeStruct((NPAD, wout), jnp.float32),
        )(acc, g, dinvb, b, w)

    return run


def _prep_kernel(acc_ref, g_ref, dv_ref, b_ref, gp_ref,
                 h3_ref, last_ref, widx_ref, lo_ref, hi_ref, ws_ref, ovf_ref):
    rows = lax.broadcasted_iota(jnp.int32, (NPAD, 1), 0)
    t = dv_ref[:, :OUT] * (acc_ref[0] + acc_ref[1] + g_ref[...]) + b_ref[...]
    h3 = jnp.where(rows < N, t, 0.0)
    h3_ref[...] = h3
    last_ref[...] = h3[:, OUT - 1:OUT]

    cnt = gp_ref[0, :NGRAPH, 0:1] + gp_ref[1, :NGRAPH, 0:1]
    gi = lax.broadcasted_iota(jnp.int32, (NGRAPH, NGRAPH), 0)
    gj = lax.broadcasted_iota(jnp.int32, (NGRAPH, NGRAPH), 1)
    lower = jnp.where(gi > gj, 1.0, 0.0)
    starts = jnp.dot(lower, cnt, preferred_element_type=jnp.float32)
    starts_i = starts.astype(jnp.int32)
    cnt_i = cnt.astype(jnp.int32)
    ws = jnp.bitwise_and(starts_i, -8)
    lo_ref[...] = starts_i - ws
    hi_ref[...] = starts_i - ws + cnt_i
    ws_ref[...] = ws
    ovf_ref[...] = jnp.max(cnt_i, keepdims=True).reshape(1, 1) > WMAX
    lane = lax.broadcasted_iota(jnp.int32, (NGRAPH, 128), 1)
    for c in range(WC):
        widx_ref[c] = jnp.minimum(ws + c * 128 + lane, NPAD - 1)


def _prep(acc, g, dinvb, b, gparts):
    return pl.pallas_call(
        _prep_kernel,
        in_specs=[
            pl.BlockSpec((NC, NPAD, OUT), lambda: (0, 0, 0)),
            pl.BlockSpec((NPAD, OUT), lambda: (0, 0)),
            pl.BlockSpec((NPAD, D), lambda: (0, 0)),
            pl.BlockSpec((1, OUT), lambda: (0, 0)),
            pl.BlockSpec((NC, 128, 16), lambda: (0, 0, 0)),
        ],
        out_specs=[pl.BlockSpec((NPAD, OUT), lambda: (0, 0)),
                   pl.BlockSpec((NPAD, 1), lambda: (0, 0)),
                   pl.BlockSpec((WC, NGRAPH, 128), lambda: (0, 0, 0)),
                   pl.BlockSpec((NGRAPH, 1), lambda: (0, 0)),
                   pl.BlockSpec((NGRAPH, 1), lambda: (0, 0)),
                   pl.BlockSpec((NGRAPH, 1), lambda: (0, 0)),
                   pl.BlockSpec((1, 1), lambda: (0, 0))],
        out_shape=[jax.ShapeDtypeStruct((NPAD, OUT), jnp.float32),
                   jax.ShapeDtypeStruct((NPAD, 1), jnp.float32),
                   jax.ShapeDtypeStruct((WC, NGRAPH, 128), jnp.int32),
                   jax.ShapeDtypeStruct((NGRAPH, 1), jnp.int32),
                   jax.ShapeDtypeStruct((NGRAPH, 1), jnp.int32),
                   jax.ShapeDtypeStruct((NGRAPH, 1), jnp.int32),
                   jax.ShapeDtypeStruct((1, 1), jnp.bool_)],
    )(acc, g, dinvb, b, gparts)


def _select_kernel(win_ref, lo_ref, hi_ref, ws_ref, idx_ref, scores_ref):
    lane = lax.broadcasted_iota(jnp.int32, (NGRAPH, WIN), 1)
    valid = (lane >= lo_ref[...]) & (lane < hi_ref[...])
    scores_ref[...] = jnp.where(valid, win_ref[...], NEG)
    tcol = lax.broadcasted_iota(jnp.int32, (NGRAPH, 32), 1)

    def body(t, carry):
        prev_ix, acc_idx = carry
        s = jnp.where(lane == prev_ix, NEG, scores_ref[...])
        scores_ref[...] = s
        m = jnp.max(s, axis=1, keepdims=True)
        cand = jnp.where(s >= m, lane, BIGI)
        ix = jnp.min(cand, axis=1, keepdims=True)
        node = jnp.where(m > -1e38, ws_ref[...] + ix, N)
        acc_idx = jnp.where(tcol == t, node, acc_idx)
        return ix, acc_idx

    _, acc_idx = lax.fori_loop(
        0, 32, body,
        (jnp.full((NGRAPH, 1), -1, jnp.int32),
         jnp.zeros((NGRAPH, 32), jnp.int32)))
    idx_ref[...] = acc_idx


def _select(win, lo, hi, ws):
    return pl.pallas_call(
        _select_kernel,
        in_specs=[
            pl.BlockSpec((NGRAPH, WIN), lambda: (0, 0)),
            pl.BlockSpec((NGRAPH, 1), lambda: (0, 0)),
            pl.BlockSpec((NGRAPH, 1), lambda: (0, 0)),
            pl.BlockSpec((NGRAPH, 1), lambda: (0, 0)),
        ],
        out_specs=pl.BlockSpec((NGRAPH, 32), lambda: (0, 0)),
        out_shape=jax.ShapeDtypeStruct((NGRAPH, 32), jnp.int32),
        scratch_shapes=[pltpu.VMEM((NGRAPH, WIN), jnp.float32)],
    )(win, lo, hi, ws)


def _topk_full_kernel(h3_ref, batch_ref, idx_ref, scores_ref):
    iota_g = lax.broadcasted_iota(jnp.int32, (NPAD, NGRAPH), 1)
    iota_n = lax.broadcasted_iota(jnp.int32, (NPAD, NGRAPH), 0)
    last = h3_ref[:, OUT - 1:OUT]
    eq = batch_ref[...] == iota_g
    scores_ref[...] = jnp.where(eq, jnp.broadcast_to(last, (NPAD, NGRAPH)), NEG)

    def body(t, prev_ix):
        s = jnp.where(iota_n == prev_ix, NEG, scores_ref[...])
        scores_ref[...] = s
        m = jnp.max(s, axis=0, keepdims=True)
        cand = jnp.where(s >= m, iota_n, BIGI)
        ix = jnp.min(cand, axis=0, keepdims=True)
        idx_ref[pl.ds(t, 1), :] = jnp.where(m > -1e38, ix, N)
        return ix

    lax.fori_loop(0, 32, body, jnp.full((1, NGRAPH), -1, jnp.int32))


def _topk_full(h3, batch2d):
    return pl.pallas_call(
        _topk_full_kernel,
        in_specs=[
            pl.BlockSpec((NPAD, OUT), lambda: (0, 0)),
            pl.BlockSpec((NPAD, 1), lambda: (0, 0)),
        ],
        out_specs=pl.BlockSpec((32, NGRAPH), lambda: (0, 0)),
        out_shape=jax.ShapeDtypeStruct((32, NGRAPH), jnp.int32),
        scratch_shapes=[pltpu.VMEM((NPAD, NGRAPH), jnp.float32)],
    )(h3, batch2d)


def kernel(x, edge_index, batch, W0, b0, W1, b1, W2, b2):
    xp = jnp.pad(x, ((0, NPAD - N), (0, 0)))
    npad_tail = NPAD - N
    srcs = edge_index[0].reshape(NW, NCHUNK, CHUNK)
    dsts = edge_index[1].reshape(NW, NCHUNK, CHUNK)
    batch2d = jnp.pad(batch, (0, NPAD - N), constant_values=NGRAPH)[:, None]
    z16 = jnp.zeros((RPT, 16), jnp.float32)
    z128 = jnp.zeros((RPT, D), jnp.float32)
    z64 = jnp.zeros((RPT, OUT), jnp.float32)
    ones16 = jnp.ones((CHUNK, 16), jnp.float32)

    batch_pad = jnp.concatenate(
        [batch, NGRAPH + (jnp.arange(NPAD - N, dtype=jnp.int32) % NGRAPH)])
    batch_chunks = batch_pad.reshape(NW, 4, 80)

    deg_parts, g_parts = _make_hist()(dsts, batch_chunks, z16, ones16)

    scat128 = _make_edge_scatter(D)
    g0, dinvb = _a0(xp, W0, deg_parts)
    acc0 = scat128(g0, srcs, dsts, z128)
    g1 = _make_a_mid(D)(acc0, g0, dinvb, b0.reshape(1, D), W1)
    acc1 = scat128(g1, srcs, dsts, z128)
    g2 = _make_a_mid(OUT)(acc1, g1, dinvb, b1.reshape(1, D), W2)
    acc2 = _make_edge_scatter(OUT)(g2, srcs, dsts, z64)

    h3, lastc, widx, lo, hi, ws, ovf = _prep(
        acc2, g2, dinvb, b2.reshape(1, OUT), g_parts)
    win = _make_win_gather()(lastc.reshape(NPAD), widx)

    def fast(_):
        return _select(win, lo, hi, ws)

    def slow(_):
        return _topk_full(h3, batch2d).T

    idx_gk = lax.cond(ovf[0, 0], slow, fast, 0)
    idx_flat = idx_gk[:, :K].reshape(-1)
    n_per_w = 64
    dummy = N + (jnp.arange(NW * n_per_w - NGRAPH * K, dtype=jnp.int32) % npad_tail)
    idx_pad = jnp.concatenate([idx_flat, dummy]).reshape(NW, n_per_w)
    rows = _make_row_gather(n_per_w)(h3, idx_pad)
    return rows[:NGRAPH * K].reshape(NGRAPH, K * OUT)
